# Initial kernel scaffold; baseline (speedup 1.0000x reference)
import functools

import jax
import jax.numpy as jnp
from jax import lax
from jax.experimental import pallas as pl
from jax.experimental.pallas import tpu as pltpu
from jax.experimental.pallas import tpu_sc as plsc

_B = 4096
_F = 26
_K = 64
_FK = _F * _K
_NN0 = 1024
_NN1 = 512
_EPS = 1e-5

_NC = 2
_NS = 16
_NW = _NC * _NS
_BPW = _B // _NW
_RPW = _BPW * _F
_SLOT = 128
_NSTEP = _RPW // _SLOT


def _sc_gather(x32, emb1, emb2):
    mesh = plsc.VectorSubcoreMesh(core_axis_name="c", subcore_axis_name="s")

    @functools.partial(
        pl.kernel,
        out_type=[
            jax.ShapeDtypeStruct((_B * _F, _K), jnp.float32),
            jax.ShapeDtypeStruct((_NW, _NSTEP, _SLOT, 1), jnp.float32),
        ],
        mesh=mesh,
        scratch_types=[
            pltpu.VMEM((_NSTEP, _SLOT), jnp.int32),
            pltpu.VMEM((_SLOT, _K), jnp.float32),
            pltpu.VMEM((_SLOT, _K), jnp.float32),
            pltpu.VMEM((_NSTEP, _SLOT, 1), jnp.float32),
            pltpu.SemaphoreType.DMA,
            pltpu.SemaphoreType.DMA,
            pltpu.SemaphoreType.DMA,
            pltpu.SemaphoreType.DMA,
            pltpu.SemaphoreType.DMA,
        ],
    )
    def gather_k(x_hbm, emb1_hbm, emb2_hbm, nn_hbm, e1_hbm,
                 idx_v, slot0, slot1, e1_v,
                 s_in0, s_in1, s_out0, s_out1, s_e1):
        w = lax.axis_index("s") * _NC + lax.axis_index("c")
        base = w * _RPW
        pltpu.sync_copy(x_hbm.at[w], idx_v)
        slots = (slot0, slot1)
        sin = (s_in0, s_in1)
        sout = (s_out0, s_out1)

        for b in range(2):
            pltpu.async_copy(emb2_hbm.at[idx_v.at[b]], slots[b], sin[b])

        def step(g, carry):
            for b in range(2):
                j = g * 2 + b
                dst = nn_hbm.at[pl.ds(base + j * _SLOT, _SLOT)]
                pltpu.async_copy(emb1_hbm.at[idx_v.at[j]], e1_v.at[j], s_e1)
                pltpu.make_async_copy(emb2_hbm.at[idx_v.at[j]], slots[b], sin[b]).wait()
                pltpu.async_copy(slots[b], dst, sout[b])
                pltpu.make_async_copy(slots[b], dst, sout[b]).wait()
                pltpu.async_copy(emb2_hbm.at[idx_v.at[j + 2]], slots[b], sin[b])
            return carry

        lax.fori_loop(0, _NSTEP // 2 - 1, step, 0)

        for b in range(2):
            j = _NSTEP - 2 + b
            dst = nn_hbm.at[pl.ds(base + j * _SLOT, _SLOT)]
            pltpu.async_copy(emb1_hbm.at[idx_v.at[j]], e1_v.at[j], s_e1)
            pltpu.make_async_copy(emb2_hbm.at[idx_v.at[j]], slots[b], sin[b]).wait()
            pltpu.async_copy(slots[b], dst, sout[b])
            pltpu.make_async_copy(slots[b], dst, sout[b]).wait()

        pltpu.make_async_copy(e1_hbm.at[w], e1_v, s_e1).wait()
        pltpu.sync_copy(e1_v, e1_hbm.at[w])

    return gather_k(x32, emb1, emb2)


def _tc_mlp(nn, e1, bias, g1, b1, be1, g2, b2, be2, W1, W2):
    BT = 512
    grid = (_B // BT,)
    inv = float((1.0 + _EPS) ** -0.5)

    def mlp_k(nn_ref, e1_ref, bias_ref, g1_ref, b1_ref, be1_ref,
              g2_ref, b2_ref, be2_ref, W1_ref, W2_ref, out_ref):
        nnb = nn_ref[...]
        fm1 = jnp.sum(e1_ref[...], axis=1)
        ii = lax.broadcasted_iota(jnp.int32, (_FK, _K), 0)
        jj = lax.broadcasted_iota(jnp.int32, (_FK, _K), 1)
        S = jnp.where(ii % _K == jj, 1.0, 0.0).astype(jnp.float32)
        sum_f = jnp.dot(nnb, S)
        fm2 = 0.5 * (jnp.sum(sum_f * sum_f, axis=1) - jnp.sum(nnb * nnb, axis=1))
        a1 = g1_ref[...] * inv
        c1 = b1_ref[...] * a1 + be1_ref[...]
        h = lax.dot_general(nnb, W1_ref[...], (((1,), (1,)), ((), ())))
        h = jnp.maximum(h * a1 + c1, 0.0)
        a2 = g2_ref[...] * inv
        c2 = b2_ref[...] * a2 + be2_ref[...]
        h = lax.dot_general(h, W2_ref[...], (((1,), (1,)), ((), ())))
        h = jnp.maximum(h * a2 + c2, 0.0)
        out_ref[...] = fm1 + fm2 + jnp.sum(h, axis=1) + bias_ref[0]

    return pl.pallas_call(
        mlp_k,
        grid=grid,
        in_specs=[
            pl.BlockSpec((BT, _FK), lambda i: (i, 0)),
            pl.BlockSpec((BT, _F), lambda i: (i, 0)),
            pl.BlockSpec(memory_space=pltpu.SMEM),
            pl.BlockSpec((1, _NN0), lambda i: (0, 0)),
            pl.BlockSpec((1, _NN0), lambda i: (0, 0)),
            pl.BlockSpec((1, _NN0), lambda i: (0, 0)),
            pl.BlockSpec((1, _NN1), lambda i: (0, 0)),
            pl.BlockSpec((1, _NN1), lambda i: (0, 0)),
            pl.BlockSpec((1, _NN1), lambda i: (0, 0)),
            pl.BlockSpec((_NN0, _FK), lambda i: (0, 0)),
            pl.BlockSpec((_NN1, _NN0), lambda i: (0, 0)),
        ],
        out_specs=pl.BlockSpec((BT,), lambda i: (i,)),
        out_shape=jax.ShapeDtypeStruct((_B,), jnp.float32),
        compiler_params=pltpu.CompilerParams(
            dimension_semantics=("arbitrary",),
        ),
    )(nn, e1, bias, g1, b1, be1, g2, b2, be2, W1, W2)


def kernel(x, bias, emb1, emb2, W1, b1, g1, be1, W2, b2, g2, be2):
    xr = x.reshape(_NW, _NSTEP, _SLOT)
    nn, e1v = _sc_gather(xr, emb1, emb2)
    nn2 = nn.reshape(_B, _FK)
    e1m = e1v.reshape(_B, _F)
    return _tc_mlp(
        nn2, e1m, bias,
        g1.reshape(1, -1), b1.reshape(1, -1), be1.reshape(1, -1),
        g2.reshape(1, -1), b2.reshape(1, -1), be2.reshape(1, -1),
        W1, W2,
    )


# speedup vs baseline: 1.4003x; 1.4003x over previous
import functools

import jax
import jax.numpy as jnp
from jax import lax
from jax.experimental import pallas as pl
from jax.experimental.pallas import tpu as pltpu
from jax.experimental.pallas import tpu_sc as plsc

_B = 4096
_F = 26
_K = 64
_FK = _F * _K
_NN0 = 1024
_NN1 = 512
_EPS = 1e-5

_NC = 2
_NS = 16
_NW = _NC * _NS
_BPW = _B // _NW
_RPW = _BPW * _F
_SLOT = 128
_NSTEP = _RPW // _SLOT


def _sc_gather(x32, emb1, emb2):
    mesh = plsc.VectorSubcoreMesh(core_axis_name="c", subcore_axis_name="s")

    @functools.partial(
        pl.kernel,
        out_type=[
            jax.ShapeDtypeStruct((_B * _F, _K), jnp.float32),
            jax.ShapeDtypeStruct((_NW, _NSTEP, _SLOT), jnp.float32),
        ],
        mesh=mesh,
        compiler_params=pltpu.CompilerParams(use_tc_tiling_on_sc=False),
        scratch_types=[
            pltpu.VMEM((_NSTEP, _SLOT), jnp.int32),
            pltpu.VMEM((_SLOT, _K), jnp.float32),
            pltpu.VMEM((_NSTEP, _SLOT), jnp.float32),
            pltpu.SemaphoreType.DMA,
            pltpu.SemaphoreType.DMA,
        ],
    )
    def gather_k(x_hbm, emb1_hbm, emb2_hbm, nn_hbm, e1_hbm,
                 idx_v, slot0, e1_v, s_in, s_e1):
        w = lax.axis_index("s") * _NC + lax.axis_index("c")
        base = w * _RPW
        pltpu.sync_copy(x_hbm.at[w], idx_v)

        def step(j, carry):
            pltpu.async_copy(emb2_hbm.at[idx_v.at[j]], slot0, s_in)
            pltpu.async_copy(emb1_hbm.at[idx_v.at[j]], e1_v.at[j], s_e1)
            pltpu.make_async_copy(emb1_hbm.at[idx_v.at[j]], e1_v.at[j], s_e1).wait()
            pltpu.make_async_copy(emb2_hbm.at[idx_v.at[j]], slot0, s_in).wait()
            pltpu.sync_copy(slot0, nn_hbm.at[pl.ds(base + j * _SLOT, _SLOT)])
            return carry

        lax.fori_loop(0, _NSTEP, step, 0)
        pltpu.sync_copy(e1_v, e1_hbm.at[w])

    return gather_k(x32, emb1, emb2)


def _tc_mlp(nn, e1, bias, g1, b1, be1, g2, b2, be2, W1, W2):
    BT = 512
    grid = (_B // BT,)
    inv = float((1.0 + _EPS) ** -0.5)

    def mlp_k(nn_ref, e1_ref, bias_ref, g1_ref, b1_ref, be1_ref,
              g2_ref, b2_ref, be2_ref, W1_ref, W2_ref, out_ref):
        nnb = nn_ref[...]
        fm1 = jnp.sum(e1_ref[...], axis=1)
        ii = lax.broadcasted_iota(jnp.int32, (_FK, _K), 0)
        jj = lax.broadcasted_iota(jnp.int32, (_FK, _K), 1)
        S = jnp.where(ii % _K == jj, 1.0, 0.0).astype(jnp.float32)
        sum_f = jnp.dot(nnb, S)
        fm2 = 0.5 * (jnp.sum(sum_f * sum_f, axis=1) - jnp.sum(nnb * nnb, axis=1))
        a1 = g1_ref[...] * inv
        c1 = b1_ref[...] * a1 + be1_ref[...]
        h = lax.dot_general(nnb, W1_ref[...], (((1,), (1,)), ((), ())))
        h = jnp.maximum(h * a1 + c1, 0.0)
        a2 = g2_ref[...] * inv
        c2 = b2_ref[...] * a2 + be2_ref[...]
        h = lax.dot_general(h, W2_ref[...], (((1,), (1,)), ((), ())))
        h = jnp.maximum(h * a2 + c2, 0.0)
        out_ref[...] = fm1 + fm2 + jnp.sum(h, axis=1) + bias_ref[0]

    return pl.pallas_call(
        mlp_k,
        grid=grid,
        in_specs=[
            pl.BlockSpec((BT, _FK), lambda i: (i, 0)),
            pl.BlockSpec((BT, _F), lambda i: (i, 0)),
            pl.BlockSpec(memory_space=pltpu.SMEM),
            pl.BlockSpec((1, _NN0), lambda i: (0, 0)),
            pl.BlockSpec((1, _NN0), lambda i: (0, 0)),
            pl.BlockSpec((1, _NN0), lambda i: (0, 0)),
            pl.BlockSpec((1, _NN1), lambda i: (0, 0)),
            pl.BlockSpec((1, _NN1), lambda i: (0, 0)),
            pl.BlockSpec((1, _NN1), lambda i: (0, 0)),
            pl.BlockSpec((_NN0, _FK), lambda i: (0, 0)),
            pl.BlockSpec((_NN1, _NN0), lambda i: (0, 0)),
        ],
        out_specs=pl.BlockSpec((BT,), lambda i: (i,)),
        out_shape=jax.ShapeDtypeStruct((_B,), jnp.float32),
        compiler_params=pltpu.CompilerParams(
            dimension_semantics=("arbitrary",),
        ),
    )(nn, e1, bias, g1, b1, be1, g2, b2, be2, W1, W2)


def kernel(x, bias, emb1, emb2, W1, b1, g1, be1, W2, b2, g2, be2):
    xr = x.reshape(_NW, _NSTEP, _SLOT)
    nn, e1v = _sc_gather(xr, emb1.reshape(-1), emb2)
    nn2 = nn.reshape(_B, _FK)
    e1m = e1v.reshape(_B, _F)
    return _tc_mlp(
        nn2, e1m, bias,
        g1.reshape(1, -1), b1.reshape(1, -1), be1.reshape(1, -1),
        g2.reshape(1, -1), b2.reshape(1, -1), be2.reshape(1, -1),
        W1, W2,
    )


# speedup vs baseline: 1.7512x; 1.2506x over previous
import functools

import jax
import jax.numpy as jnp
from jax import lax
from jax.experimental import pallas as pl
from jax.experimental.pallas import tpu as pltpu
from jax.experimental.pallas import tpu_sc as plsc

_B = 4096
_F = 26
_K = 64
_FK = _F * _K
_CT = _FK // 128
_NN0 = 1024
_NN1 = 512
_EPS = 1e-5

_NC = 2
_NS = 16
_NW = _NC * _NS
_BPW = _B // _NW
_RPW = _BPW * _F
_SLOT = 128
_NSTEP = _RPW // _SLOT


def _sc_gather(xp, xt, emb1, emb2):
    mesh = plsc.VectorSubcoreMesh(core_axis_name="c", subcore_axis_name="s")

    @functools.partial(
        pl.kernel,
        out_type=[
            jax.ShapeDtypeStruct((_CT, 2 * _B, _K), jnp.float32),
            jax.ShapeDtypeStruct((_B,), jnp.float32),
        ],
        mesh=mesh,
        compiler_params=pltpu.CompilerParams(use_tc_tiling_on_sc=False),
        scratch_types=[
            pltpu.VMEM((_NSTEP, _SLOT), jnp.int32),
            pltpu.VMEM((_F, _BPW), jnp.int32),
            pltpu.VMEM((_SLOT, _K), jnp.float32),
            pltpu.VMEM((_SLOT, _K), jnp.float32),
            pltpu.VMEM((_F, _BPW), jnp.float32),
            pltpu.VMEM((_BPW,), jnp.float32),
            pltpu.SemaphoreType.DMA,
            pltpu.SemaphoreType.DMA,
            pltpu.SemaphoreType.DMA,
            pltpu.SemaphoreType.DMA,
            pltpu.SemaphoreType.DMA,
        ],
    )
    def gather_k(xp_hbm, xt_hbm, emb1_hbm, emb2_hbm, nn_hbm, fm1_hbm,
                 idx_v, idx2_v, slot0, slot1, e1_v, fm1_v,
                 s_in0, s_in1, s_out0, s_out1, s_e1):
        w = lax.axis_index("s") * _NC + lax.axis_index("c")
        pltpu.sync_copy(xp_hbm.at[w], idx_v)
        pltpu.sync_copy(xt_hbm.at[w], idx2_v)

        def e1_fire(j, carry):
            pltpu.async_copy(emb1_hbm.at[idx2_v.at[j]], e1_v.at[j], s_e1)
            return carry

        lax.fori_loop(0, _F, e1_fire, 0)

        slots = (slot0, slot1)
        sin = (s_in0, s_in1)
        sout = (s_out0, s_out1)

        def nn_dst(j):
            c = j // 2
            return nn_hbm.at[c, pl.ds(w * 2 * _SLOT + (j % 2) * _SLOT, _SLOT)]

        for b in range(2):
            pltpu.async_copy(emb2_hbm.at[idx_v.at[b]], slots[b], sin[b])

        def step(g, carry):
            for b in range(2):
                j = g * 2 + b
                pltpu.make_async_copy(emb2_hbm.at[idx_v.at[j]], slots[b], sin[b]).wait()
                pltpu.async_copy(slots[b], nn_dst(j), sout[b])
                pltpu.make_async_copy(slots[b], nn_dst(j), sout[b]).wait()
                pltpu.async_copy(emb2_hbm.at[idx_v.at[j + 2]], slots[b], sin[b])
            return carry

        lax.fori_loop(0, _NSTEP // 2 - 1, step, 0)

        for b in range(2):
            j = _NSTEP - 2 + b
            pltpu.make_async_copy(emb2_hbm.at[idx_v.at[j]], slots[b], sin[b]).wait()
            pltpu.async_copy(slots[b], nn_dst(j), sout[b])
            pltpu.make_async_copy(slots[b], nn_dst(j), sout[b]).wait()

        def e1_drain(j, carry):
            pltpu.make_async_copy(emb1_hbm.at[idx2_v.at[j]], e1_v.at[j], s_e1).wait()
            return carry

        lax.fori_loop(0, _F, e1_drain, 0)

        for g in range(_BPW // 16):
            acc = e1_v[0, pl.ds(g * 16, 16)]
            for f in range(1, _F):
                acc = acc + e1_v[f, pl.ds(g * 16, 16)]
            fm1_v[pl.ds(g * 16, 16)] = acc
        pltpu.sync_copy(fm1_v, fm1_hbm.at[pl.ds(w * _BPW, _BPW)])

    return gather_k(xp, xt, emb1, emb2)


def _tc_mlp(nn3, fm1, bias, g1, b1, be1, g2, b2, be2, W1, W2):
    BT = 512
    grid = (_B // BT,)
    inv = float((1.0 + _EPS) ** -0.5)

    def mlp_k(nn_ref, fm1_ref, bias_ref, g1_ref, b1_ref, be1_ref,
              g2_ref, b2_ref, be2_ref, W1_ref, W2_ref, out_ref):
        fold = None
        sq = None
        acc = None
        for c in range(_CT):
            slab = nn_ref[c]
            fold = slab if fold is None else fold + slab
            s = jnp.sum(slab * slab, axis=1)
            sq = s if sq is None else sq + s
            p = lax.dot_general(slab, W1_ref[:, 128 * c:128 * (c + 1)],
                                (((1,), (1,)), ((), ())))
            acc = p if acc is None else acc + p
        sum_f = (lax.slice_in_dim(fold, 0, _K, axis=1)
                 + lax.slice_in_dim(fold, _K, 2 * _K, axis=1))
        fm2 = 0.5 * (jnp.sum(sum_f * sum_f, axis=1) - sq)
        a1 = g1_ref[...] * inv
        c1 = b1_ref[...] * a1 + be1_ref[...]
        h = jnp.maximum(acc * a1 + c1, 0.0)
        a2 = g2_ref[...] * inv
        c2 = b2_ref[...] * a2 + be2_ref[...]
        h = lax.dot_general(h, W2_ref[...], (((1,), (1,)), ((), ())))
        h = jnp.maximum(h * a2 + c2, 0.0)
        out_ref[...] = fm1_ref[...] + fm2 + jnp.sum(h, axis=1) + bias_ref[0]

    return pl.pallas_call(
        mlp_k,
        grid=grid,
        in_specs=[
            pl.BlockSpec((_CT, BT, 128), lambda i: (0, i, 0)),
            pl.BlockSpec((BT,), lambda i: (i,)),
            pl.BlockSpec(memory_space=pltpu.SMEM),
            pl.BlockSpec((1, _NN0), lambda i: (0, 0)),
            pl.BlockSpec((1, _NN0), lambda i: (0, 0)),
            pl.BlockSpec((1, _NN0), lambda i: (0, 0)),
            pl.BlockSpec((1, _NN1), lambda i: (0, 0)),
            pl.BlockSpec((1, _NN1), lambda i: (0, 0)),
            pl.BlockSpec((1, _NN1), lambda i: (0, 0)),
            pl.BlockSpec((_NN0, _FK), lambda i: (0, 0)),
            pl.BlockSpec((_NN1, _NN0), lambda i: (0, 0)),
        ],
        out_specs=pl.BlockSpec((BT,), lambda i: (i,)),
        out_shape=jax.ShapeDtypeStruct((_B,), jnp.float32),
        compiler_params=pltpu.CompilerParams(
            dimension_semantics=("arbitrary",),
        ),
    )(nn3, fm1, bias, g1, b1, be1, g2, b2, be2, W1, W2)


def kernel(x, bias, emb1, emb2, W1, b1, g1, be1, W2, b2, g2, be2):
    xp = (x.reshape(_NW, 2, 64, _CT, 2)
           .transpose(0, 3, 1, 2, 4)
           .reshape(_NW, _NSTEP, _SLOT))
    xt = x.reshape(_NW, _BPW, _F).transpose(0, 2, 1)
    nn, fm1 = _sc_gather(xp, xt, emb1.reshape(-1), emb2)
    nn3 = nn.reshape(_CT, _B, 128)
    return _tc_mlp(
        nn3, fm1, bias,
        g1.reshape(1, -1), b1.reshape(1, -1), be1.reshape(1, -1),
        g2.reshape(1, -1), b2.reshape(1, -1), be2.reshape(1, -1),
        W1, W2,
    )
